# baseline (speedup 1.0000x reference)
import functools

import jax
import jax.numpy as jnp
from jax.experimental import pallas as pl
from jax.experimental.pallas import tpu as pltpu

_BN_EPS = 1e-5


def _pick_tile(n, cands=(2048, 1024, 512, 256, 128, 64, 32, 16, 8)):
    for c in cands:
        if n % c == 0:
            return c
    return n


def _mm_stats_kernel(x_ref, w_ref, y_ref, sum_ref, sumsq_ref):
    x = x_ref[...].astype(jnp.bfloat16)
    y = jnp.dot(x, w_ref[...], preferred_element_type=jnp.float32)
    y_ref[...] = y.astype(jnp.bfloat16)
    y3 = y.reshape(-1, 8, y.shape[-1])
    sum_ref[...] = jnp.sum(y3, axis=0)
    sumsq_ref[...] = jnp.sum(y3 * y3, axis=0)


def _affine_relu_kernel(y_ref, sum_ref, sumsq_ref, gamma_ref, beta_ref, o_ref,
                        *, inv_n):
    sum_c = jnp.sum(sum_ref[...], axis=0, keepdims=True)
    sumsq_c = jnp.sum(sumsq_ref[...], axis=0, keepdims=True)
    mean = sum_c * inv_n
    var = jnp.maximum(sumsq_c * inv_n - mean * mean, 0.0)
    scale = gamma_ref[...] * jax.lax.rsqrt(var + _BN_EPS)
    shift = beta_ref[...] - mean * scale
    y = y_ref[...].astype(jnp.float32)
    o_ref[...] = jnp.maximum(y * scale + shift, 0.0)


@jax.jit
def _forward(src, w, gamma, beta):
    B, T, Din = src.shape
    E = w.shape[1]
    N = B * T
    x2d = src.reshape(N, Din)
    w_bf = w.astype(jnp.bfloat16)

    tn = _pick_tile(N)
    num_n = N // tn

    y_bf, sum_p, sumsq_p = pl.pallas_call(
        _mm_stats_kernel,
        out_shape=(
            jax.ShapeDtypeStruct((N, E), jnp.bfloat16),
            jax.ShapeDtypeStruct((num_n * 8, E), jnp.float32),
            jax.ShapeDtypeStruct((num_n * 8, E), jnp.float32),
        ),
        grid=(num_n,),
        in_specs=[
            pl.BlockSpec((tn, Din), lambda n: (n, 0)),
            pl.BlockSpec((Din, E), lambda n: (0, 0)),
        ],
        out_specs=[
            pl.BlockSpec((tn, E), lambda n: (n, 0)),
            pl.BlockSpec((8, E), lambda n: (n, 0)),
            pl.BlockSpec((8, E), lambda n: (n, 0)),
        ],
        compiler_params=pltpu.CompilerParams(
            dimension_semantics=("parallel",),
            vmem_limit_bytes=48 * 1024 * 1024,
        ),
    )(x2d, w_bf)

    tm = _pick_tile(N, cands=(4096, 2048, 1024, 512, 256, 128, 64, 32, 16, 8))
    num_m = N // tm
    out2d = pl.pallas_call(
        functools.partial(_affine_relu_kernel, inv_n=1.0 / N),
        out_shape=jax.ShapeDtypeStruct((N, E), src.dtype),
        grid=(num_m,),
        in_specs=[
            pl.BlockSpec((tm, E), lambda m: (m, 0)),
            pl.BlockSpec((num_n * 8, E), lambda m: (0, 0)),
            pl.BlockSpec((num_n * 8, E), lambda m: (0, 0)),
            pl.BlockSpec((1, E), lambda m: (0, 0)),
            pl.BlockSpec((1, E), lambda m: (0, 0)),
        ],
        out_specs=pl.BlockSpec((tm, E), lambda m: (m, 0)),
        compiler_params=pltpu.CompilerParams(
            dimension_semantics=("parallel",),
            vmem_limit_bytes=48 * 1024 * 1024,
        ),
    )(y_bf, sum_p, sumsq_p, gamma, beta)

    return out2d.reshape(B, T, E)


def kernel(src, w, b, gamma, beta):
    del b
    return _forward(src, w, gamma, beta)


# speedup vs baseline: 1.0178x; 1.0178x over previous
import functools

import jax
import jax.numpy as jnp
from jax.experimental import pallas as pl
from jax.experimental.pallas import tpu as pltpu

_BN_EPS = 1e-5


def _pick_tile(n, cands=(4096, 2048, 1024, 512, 256, 128, 64, 32, 16, 8)):
    for c in cands:
        if n % c == 0:
            return c
    return n


def _mm_stats_kernel(x_ref, w_ref, y_ref, sum_ref, sumsq_ref):
    x = x_ref[...].astype(jnp.bfloat16)
    y = jnp.dot(x, w_ref[...], preferred_element_type=jnp.float32)
    y_ref[...] = y.astype(jnp.bfloat16)
    y3 = y.reshape(-1, 8, y.shape[-1])
    sum_ref[...] = jnp.sum(y3, axis=0)
    sumsq_ref[...] = jnp.sum(y3 * y3, axis=0)


def _affine_relu_kernel(y_ref, sum_ref, sumsq_ref, gamma_ref, beta_ref, o_ref,
                        *, inv_n):
    sum_c = jnp.sum(sum_ref[...], axis=0, keepdims=True)
    sumsq_c = jnp.sum(sumsq_ref[...], axis=0, keepdims=True)
    mean = sum_c * inv_n
    var = jnp.maximum(sumsq_c * inv_n - mean * mean, 0.0)
    scale = gamma_ref[...] * jax.lax.rsqrt(var + _BN_EPS)
    shift = beta_ref[...] - mean * scale
    y = y_ref[...].astype(jnp.float32)
    o_ref[...] = jnp.maximum(y * scale + shift, 0.0)


@jax.jit
def _forward(src, w, gamma, beta):
    B, T, Din = src.shape
    E = w.shape[1]
    N = B * T
    x2d = src.reshape(N, Din)
    w_bf = w.astype(jnp.bfloat16)

    tn = _pick_tile(N)
    num_n = N // tn

    y_bf, sum_p, sumsq_p = pl.pallas_call(
        _mm_stats_kernel,
        out_shape=(
            jax.ShapeDtypeStruct((N, E), jnp.bfloat16),
            jax.ShapeDtypeStruct((num_n * 8, E), jnp.float32),
            jax.ShapeDtypeStruct((num_n * 8, E), jnp.float32),
        ),
        grid=(num_n,),
        in_specs=[
            pl.BlockSpec((tn, Din), lambda n: (n, 0)),
            pl.BlockSpec((Din, E), lambda n: (0, 0)),
        ],
        out_specs=[
            pl.BlockSpec((tn, E), lambda n: (n, 0)),
            pl.BlockSpec((8, E), lambda n: (n, 0)),
            pl.BlockSpec((8, E), lambda n: (n, 0)),
        ],
        compiler_params=pltpu.CompilerParams(
            dimension_semantics=("parallel",),
            vmem_limit_bytes=48 * 1024 * 1024,
        ),
    )(x2d, w_bf)

    tm = _pick_tile(N, cands=(4096, 2048, 1024, 512, 256, 128, 64, 32, 16, 8))
    num_m = N // tm
    out2d = pl.pallas_call(
        functools.partial(_affine_relu_kernel, inv_n=1.0 / N),
        out_shape=jax.ShapeDtypeStruct((N, E), src.dtype),
        grid=(num_m,),
        in_specs=[
            pl.BlockSpec((tm, E), lambda m: (m, 0)),
            pl.BlockSpec((num_n * 8, E), lambda m: (0, 0)),
            pl.BlockSpec((num_n * 8, E), lambda m: (0, 0)),
            pl.BlockSpec((1, E), lambda m: (0, 0)),
            pl.BlockSpec((1, E), lambda m: (0, 0)),
        ],
        out_specs=pl.BlockSpec((tm, E), lambda m: (m, 0)),
        compiler_params=pltpu.CompilerParams(
            dimension_semantics=("parallel",),
            vmem_limit_bytes=48 * 1024 * 1024,
        ),
    )(y_bf, sum_p, sumsq_p, gamma, beta)

    return out2d.reshape(B, T, E)


def kernel(src, w, b, gamma, beta):
    del b
    return _forward(src, w, gamma, beta)


# speedup vs baseline: 1.0243x; 1.0064x over previous
import functools

import jax
import jax.numpy as jnp
from jax.experimental import pallas as pl
from jax.experimental.pallas import tpu as pltpu

_BN_EPS = 1e-5


def _pick_tile(n, cands=(4096, 2048, 1024, 512, 256, 128, 64, 32, 16, 8)):
    for c in cands:
        if n % c == 0:
            return c
    return n


def _mm_stats_kernel(x_ref, w_ref, y_ref, sum_ref, sumsq_ref):
    x = x_ref[...].astype(jnp.bfloat16)
    y = jnp.dot(x, w_ref[...], preferred_element_type=jnp.float32)
    y_ref[...] = y.astype(jnp.bfloat16)
    y3 = y.reshape(-1, 8, y.shape[-1])
    sum_ref[...] = jnp.sum(y3, axis=0)
    sumsq_ref[...] = jnp.sum(y3 * y3, axis=0)


def _affine_relu_kernel(y_ref, sum_ref, sumsq_ref, gamma_ref, beta_ref, o_ref,
                        *, inv_n):
    sum_c = jnp.sum(sum_ref[...], axis=0, keepdims=True)
    sumsq_c = jnp.sum(sumsq_ref[...], axis=0, keepdims=True)
    mean = sum_c * inv_n
    var = jnp.maximum(sumsq_c * inv_n - mean * mean, 0.0)
    scale = gamma_ref[...] * jax.lax.rsqrt(var + _BN_EPS)
    shift = beta_ref[...] - mean * scale
    y = y_ref[...].astype(jnp.float32)
    o_ref[...] = jnp.maximum(y * scale + shift, 0.0)


@jax.jit
def _forward(src, w, gamma, beta):
    B, T, Din = src.shape
    E = w.shape[1]
    N = B * T
    x2d = src.reshape(N, Din)
    w_bf = w.astype(jnp.bfloat16)

    tn = _pick_tile(N)
    num_n = N // tn

    y_bf, sum_p, sumsq_p = pl.pallas_call(
        _mm_stats_kernel,
        out_shape=(
            jax.ShapeDtypeStruct((N, E), jnp.bfloat16),
            jax.ShapeDtypeStruct((num_n * 8, E), jnp.float32),
            jax.ShapeDtypeStruct((num_n * 8, E), jnp.float32),
        ),
        grid=(num_n,),
        in_specs=[
            pl.BlockSpec((tn, Din), lambda n: (n, 0)),
            pl.BlockSpec((Din, E), lambda n: (0, 0)),
        ],
        out_specs=[
            pl.BlockSpec((tn, E), lambda n: (n, 0)),
            pl.BlockSpec((8, E), lambda n: (n, 0)),
            pl.BlockSpec((8, E), lambda n: (n, 0)),
        ],
        compiler_params=pltpu.CompilerParams(
            dimension_semantics=("parallel",),
            vmem_limit_bytes=48 * 1024 * 1024,
        ),
    )(x2d, w_bf)

    tm = _pick_tile(N, cands=(8192, 4096, 2048, 1024, 512, 256, 128, 64, 32, 16, 8))
    num_m = N // tm
    out2d = pl.pallas_call(
        functools.partial(_affine_relu_kernel, inv_n=1.0 / N),
        out_shape=jax.ShapeDtypeStruct((N, E), src.dtype),
        grid=(num_m,),
        in_specs=[
            pl.BlockSpec((tm, E), lambda m: (m, 0)),
            pl.BlockSpec((num_n * 8, E), lambda m: (0, 0)),
            pl.BlockSpec((num_n * 8, E), lambda m: (0, 0)),
            pl.BlockSpec((1, E), lambda m: (0, 0)),
            pl.BlockSpec((1, E), lambda m: (0, 0)),
        ],
        out_specs=pl.BlockSpec((tm, E), lambda m: (m, 0)),
        compiler_params=pltpu.CompilerParams(
            dimension_semantics=("parallel",),
            vmem_limit_bytes=56 * 1024 * 1024,
        ),
    )(y_bf, sum_p, sumsq_p, gamma, beta)

    return out2d.reshape(B, T, E)


def kernel(src, w, b, gamma, beta):
    del b
    return _forward(src, w, gamma, beta)


# speedup vs baseline: 1.3786x; 1.3459x over previous
import functools

import jax
import jax.numpy as jnp
from jax.experimental import pallas as pl
from jax.experimental.pallas import tpu as pltpu

_BN_EPS = 1e-5


def _pick_tile(n, cands=(2048, 1024, 512, 256, 128, 64, 32, 16, 8)):
    for c in cands:
        if n % c == 0:
            return c
    return n


def _fused_kernel(x_ref, w_ref, gamma_ref, beta_ref, o_ref,
                  y_scr, sum_scr, sumsq_scr, *, num_n, inv_n):
    j = pl.program_id(0)

    @pl.when(j == 0)
    def _init():
        sum_scr[...] = jnp.zeros_like(sum_scr)
        sumsq_scr[...] = jnp.zeros_like(sumsq_scr)

    @pl.when(j < num_n)
    def _compute_pass():
        x = x_ref[...].astype(jnp.bfloat16)
        w = w_ref[...].astype(jnp.bfloat16)
        y = jnp.dot(x, w, preferred_element_type=jnp.float32)
        y_scr[j] = y.astype(jnp.bfloat16)
        y3 = y.reshape(-1, 8, y.shape[-1])
        sum_scr[...] += jnp.sum(y3, axis=0)
        sumsq_scr[...] += jnp.sum(y3 * y3, axis=0)

    @pl.when(j >= num_n)
    def _apply_pass():
        sum_c = jnp.sum(sum_scr[...], axis=0, keepdims=True)
        sumsq_c = jnp.sum(sumsq_scr[...], axis=0, keepdims=True)
        mean = sum_c * inv_n
        var = jnp.maximum(sumsq_c * inv_n - mean * mean, 0.0)
        scale = gamma_ref[...] * jax.lax.rsqrt(var + _BN_EPS)
        shift = beta_ref[...] - mean * scale
        y = y_scr[j - num_n].astype(jnp.float32)
        o_ref[...] = jnp.maximum(y * scale + shift, 0.0)


@jax.jit
def _forward(src, w, gamma, beta):
    B, T, Din = src.shape
    E = w.shape[1]
    N = B * T
    x2d = src.reshape(N, Din)

    tn = _pick_tile(N)
    num_n = N // tn

    out2d = pl.pallas_call(
        functools.partial(_fused_kernel, num_n=num_n, inv_n=1.0 / N),
        out_shape=jax.ShapeDtypeStruct((N, E), src.dtype),
        grid=(2 * num_n,),
        in_specs=[
            pl.BlockSpec((tn, Din), lambda j: (jnp.minimum(j, num_n - 1), 0)),
            pl.BlockSpec((Din, E), lambda j: (0, 0)),
            pl.BlockSpec((1, E), lambda j: (0, 0)),
            pl.BlockSpec((1, E), lambda j: (0, 0)),
        ],
        out_specs=pl.BlockSpec(
            (tn, E), lambda j: (jnp.maximum(j - num_n, 0), 0)),
        scratch_shapes=[
            pltpu.VMEM((num_n, tn, E), jnp.bfloat16),
            pltpu.VMEM((8, E), jnp.float32),
            pltpu.VMEM((8, E), jnp.float32),
        ],
        compiler_params=pltpu.CompilerParams(
            dimension_semantics=("arbitrary",),
            vmem_limit_bytes=56 * 1024 * 1024,
        ),
    )(x2d, w, gamma, beta)

    return out2d.reshape(B, T, E)


def kernel(src, w, b, gamma, beta):
    del b
    return _forward(src, w, gamma, beta)
